# baseline (speedup 1.0000x reference)
import functools

import jax
import jax.numpy as jnp
from jax import lax
from jax.experimental import pallas as pl
from jax.experimental.pallas import tpu as pltpu
from jax.experimental.pallas import tpu_sc as plsc

BATCH = 16384
HIST = 50
EMBED_DIM = 16
VOCAB = 1000000

NC = 2
NS = 16
NW = NC * NS
B_PER_W = BATCH // NW
CB = 128
NCHUNK = B_PER_W // CB

TC_BK = 131072


def _tc_body(w_ref, tt_ref, t_ref):
    t_ref[...] = jnp.dot(w_ref[...], tt_ref[...],
                         preferred_element_type=jnp.float32)[0]


@jax.jit
def _fold_table(tt, wrow):
    grid = (VOCAB + TC_BK - 1) // TC_BK
    return pl.pallas_call(
        _tc_body,
        grid=(grid,),
        in_specs=[
            pl.BlockSpec((1, EMBED_DIM), lambda i: (0, 0)),
            pl.BlockSpec((EMBED_DIM, TC_BK), lambda i: (0, i)),
        ],
        out_specs=pl.BlockSpec((TC_BK,), lambda i: (i,)),
        out_shape=jax.ShapeDtypeStruct((VOCAB,), jnp.float32),
    )(wrow, tt)


def _sc_body(xt_hbm, t_hbm, bv_hbm, out_hbm,
             idx_v, vals_v, out_v, bv_v, t_sh, sem, isem):
    cid = lax.axis_index("c")
    sid = lax.axis_index("s")
    wid = sid * NC + cid

    idescs = []
    for c in range(NCHUNK):
        col0 = wid * B_PER_W + c * CB
        idescs.append(
            pltpu.async_copy(xt_hbm.at[:, pl.ds(col0, CB)], idx_v.at[c], isem)
        )
    pltpu.sync_copy(bv_hbm, bv_v)
    bvec = bv_v[...]

    for d in idescs:
        d.wait()

    def chunk_body(c, carry):
        descs = []
        for l in range(HIST):
            descs.append(
                pltpu.async_copy(t_hbm.at[idx_v.at[c, l]], vals_v.at[l], sem)
            )
        for d in descs:
            d.wait()

        for j in range(CB // 16):
            acc0 = vals_v[0, pl.ds(j * 16, 16)]
            acc1 = vals_v[1, pl.ds(j * 16, 16)]
            for l in range(2, HIST, 2):
                acc0 = acc0 + vals_v[l, pl.ds(j * 16, 16)]
                acc1 = acc1 + vals_v[l + 1, pl.ds(j * 16, 16)]
            zv = (acc0 + acc1) + bvec
            s = 1.0 / (1.0 + jnp.exp(-zv))
            r4 = (s * 1e4 + 0.5).astype(jnp.int32).astype(jnp.float32) * 1e-4
            out_v[pl.ds(c * CB + j * 16, 16)] = r4
        return carry

    lax.fori_loop(0, NCHUNK, chunk_body, 0, unroll=False)

    pltpu.sync_copy(out_v, out_hbm.at[pl.ds(wid * B_PER_W, B_PER_W)])


@jax.jit
def _embed_bag_sc(xt, t, bv):
    mesh = plsc.VectorSubcoreMesh(core_axis_name="c", subcore_axis_name="s")
    f = pl.kernel(
        _sc_body,
        out_type=jax.ShapeDtypeStruct((BATCH,), jnp.float32),
        mesh=mesh,
        scratch_types=[
            pltpu.VMEM((NCHUNK, HIST, CB), jnp.int32),
            pltpu.VMEM((HIST, CB), jnp.float32),
            pltpu.VMEM((B_PER_W,), jnp.float32),
            pltpu.VMEM((EMBED_DIM,), jnp.float32),
            pltpu.VMEM_SHARED((VOCAB,), jnp.float32),
            pltpu.SemaphoreType.DMA,
            pltpu.SemaphoreType.DMA,
        ],
    )
    return f(xt, t, bv)


def kernel(x, table, W, b):
    xt = x.astype(jnp.int32).T
    tt = table.T
    wrow = (W.reshape(1, EMBED_DIM) / HIST).astype(jnp.float32)
    t = _fold_table(tt, wrow)
    bv = jnp.broadcast_to(b.astype(jnp.float32), (EMBED_DIM,))
    out = _embed_bag_sc(xt, t, bv)
    return out.reshape(BATCH, 1)
